# Initial kernel scaffold; baseline (speedup 1.0000x reference)
import jax
import jax.numpy as jnp
from jax.experimental import pallas as pl


def kernel(indices, table):
    raise NotImplementedError("write your pallas kernel here")


# speedup vs baseline: 6.3295x; 6.3295x over previous
import functools

import jax
import jax.numpy as jnp
from jax import lax
from jax.experimental import pallas as pl
from jax.experimental.pallas import tpu as pltpu
from jax.experimental.pallas import tpu_sc as plsc

_D = 128
_NW = 32
_CHUNK = 128


def _make_gather(B):
    bpw = B // _NW
    nch = bpw // _CHUNK
    mesh = plsc.VectorSubcoreMesh(core_axis_name="c", subcore_axis_name="s")

    @functools.partial(
        pl.kernel,
        mesh=mesh,
        out_type=jax.ShapeDtypeStruct((B, _D), jnp.float32),
        scratch_types=[
            pltpu.VMEM((nch, _CHUNK), jnp.int32),
            pltpu.VMEM((_CHUNK, _D), jnp.float32),
            pltpu.SemaphoreType.DMA,
        ],
    )
    def gather_kernel(idx_hbm, table_hbm, out_hbm, idx_v, rows_v, sem):
        cid = lax.axis_index("c")
        sid = lax.axis_index("s")
        wid = sid * 2 + cid
        base = wid * bpw
        pltpu.sync_copy(idx_hbm.at[wid], idx_v)

        def body(j, carry):
            pltpu.async_copy(table_hbm.at[idx_v.at[j]], rows_v, sem).wait()
            pltpu.sync_copy(rows_v, out_hbm.at[pl.ds(base + j * _CHUNK, _CHUNK)])
            return carry

        lax.fori_loop(0, nch, body, 0)

    return gather_kernel


def kernel(indices, table):
    bsz, hist = indices.shape
    B = bsz * hist
    idx = indices.astype(jnp.int32).reshape(_NW, B // _NW // _CHUNK, _CHUNK)
    out = _make_gather(B)(idx, table)
    return out.reshape(bsz, hist, _D)


# speedup vs baseline: 9.2453x; 1.4607x over previous
import functools

import jax
import jax.numpy as jnp
from jax import lax
from jax.experimental import pallas as pl
from jax.experimental.pallas import tpu as pltpu
from jax.experimental.pallas import tpu_sc as plsc

_D = 128
_NW = 32
_CHUNK = 128
_NBUF = 5
_AHEAD = 3


def _make_gather(B):
    bpw = B // _NW
    nch = bpw // _CHUNK
    assert nch % _NBUF == 0
    mesh = plsc.VectorSubcoreMesh(core_axis_name="c", subcore_axis_name="s")

    @functools.partial(
        pl.kernel,
        mesh=mesh,
        out_type=jax.ShapeDtypeStruct((B, _D), jnp.float32),
        scratch_types=[
            pltpu.VMEM((nch, _CHUNK), jnp.int32),
            pltpu.VMEM((_NBUF, _CHUNK, _D), jnp.float32),
            pltpu.SemaphoreType.DMA((_NBUF,)),
            pltpu.SemaphoreType.DMA((_NBUF,)),
        ],
    )
    def gather_kernel(idx_hbm, table_hbm, out_hbm, idx_v, rows_v, gsem, ssem):
        cid = lax.axis_index("c")
        sid = lax.axis_index("s")
        wid = sid * 2 + cid
        base = wid * bpw
        pltpu.sync_copy(idx_hbm.at[wid], idx_v)

        for b in range(_AHEAD):
            pltpu.async_copy(table_hbm.at[idx_v.at[b]], rows_v.at[b], gsem.at[b])

        def outer(p, carry):
            for b in range(_NBUF):
                j = p * _NBUF + b
                pltpu.make_async_copy(
                    table_hbm.at[idx_v.at[j]], rows_v.at[b], gsem.at[b]
                ).wait()
                pltpu.async_copy(
                    rows_v.at[b],
                    out_hbm.at[pl.ds(base + j * _CHUNK, _CHUNK)],
                    ssem.at[b],
                )
                bf = (b + _AHEAD) % _NBUF
                f = j + _AHEAD
                fprev = f - _NBUF

                @pl.when(fprev >= 0)
                def _wait_prev():
                    pltpu.make_async_copy(
                        rows_v.at[bf],
                        out_hbm.at[pl.ds(base + (fprev) * _CHUNK, _CHUNK)],
                        ssem.at[bf],
                    ).wait()

                @pl.when(f < nch)
                def _prefetch():
                    pltpu.async_copy(
                        table_hbm.at[idx_v.at[f]], rows_v.at[bf], gsem.at[bf]
                    )

            return carry

        lax.fori_loop(0, nch // _NBUF, outer, 0)

        for j_last in range(nch - (_NBUF - _AHEAD), nch):
            b = j_last % _NBUF
            pltpu.make_async_copy(
                rows_v.at[b],
                out_hbm.at[pl.ds(base + j_last * _CHUNK, _CHUNK)],
                ssem.at[b],
            ).wait()

    return gather_kernel


def kernel(indices, table):
    bsz, hist = indices.shape
    B = bsz * hist
    idx = indices.astype(jnp.int32).reshape(_NW, B // _NW // _CHUNK, _CHUNK)
    out = _make_gather(B)(idx, table)
    return out.reshape(bsz, hist, _D)


# speedup vs baseline: 9.2486x; 1.0004x over previous
import functools

import jax
import jax.numpy as jnp
from jax import lax
from jax.experimental import pallas as pl
from jax.experimental.pallas import tpu as pltpu
from jax.experimental.pallas import tpu_sc as plsc

_D = 128
_NW = 32
_CHUNK = 128
_NBUF = 5
_AHEAD = 4


def _make_gather(B):
    bpw = B // _NW
    nch = bpw // _CHUNK
    assert nch % _NBUF == 0
    mesh = plsc.VectorSubcoreMesh(core_axis_name="c", subcore_axis_name="s")

    @functools.partial(
        pl.kernel,
        mesh=mesh,
        out_type=jax.ShapeDtypeStruct((B, _D), jnp.float32),
        scratch_types=[
            pltpu.VMEM((nch, _CHUNK), jnp.int32),
            pltpu.VMEM((_NBUF, _CHUNK, _D), jnp.float32),
            pltpu.SemaphoreType.DMA((_NBUF,)),
            pltpu.SemaphoreType.DMA((_NBUF,)),
        ],
    )
    def gather_kernel(idx_hbm, table_hbm, out_hbm, idx_v, rows_v, gsem, ssem):
        cid = lax.axis_index("c")
        sid = lax.axis_index("s")
        wid = sid * 2 + cid
        base = wid * bpw
        pltpu.sync_copy(idx_hbm.at[wid], idx_v)

        for b in range(_AHEAD):
            pltpu.async_copy(table_hbm.at[idx_v.at[b]], rows_v.at[b], gsem.at[b])

        def outer(p, carry):
            for b in range(_NBUF):
                j = p * _NBUF + b
                pltpu.make_async_copy(
                    table_hbm.at[idx_v.at[j]], rows_v.at[b], gsem.at[b]
                ).wait()
                pltpu.async_copy(
                    rows_v.at[b],
                    out_hbm.at[pl.ds(base + j * _CHUNK, _CHUNK)],
                    ssem.at[b],
                )
                bf = (b + _AHEAD) % _NBUF
                f = j + _AHEAD
                fprev = f - _NBUF

                @pl.when(fprev >= 0)
                def _wait_prev():
                    pltpu.make_async_copy(
                        rows_v.at[bf],
                        out_hbm.at[pl.ds(base + (fprev) * _CHUNK, _CHUNK)],
                        ssem.at[bf],
                    ).wait()

                @pl.when(f < nch)
                def _prefetch():
                    pltpu.async_copy(
                        table_hbm.at[idx_v.at[f]], rows_v.at[bf], gsem.at[bf]
                    )

            return carry

        lax.fori_loop(0, nch // _NBUF, outer, 0)

        for j_last in range(nch - (_NBUF - _AHEAD), nch):
            b = j_last % _NBUF
            pltpu.make_async_copy(
                rows_v.at[b],
                out_hbm.at[pl.ds(base + j_last * _CHUNK, _CHUNK)],
                ssem.at[b],
            ).wait()

    return gather_kernel


def kernel(indices, table):
    bsz, hist = indices.shape
    B = bsz * hist
    idx = indices.astype(jnp.int32).reshape(_NW, B // _NW // _CHUNK, _CHUNK)
    out = _make_gather(B)(idx, table)
    return out.reshape(bsz, hist, _D)
